# baseline (speedup 1.0000x reference)
import functools

import jax
import jax.numpy as jnp
from jax import lax
from jax.experimental import pallas as pl
from jax.experimental.pallas import tpu as pltpu
from jax.experimental.pallas import tpu_sc as plsc

EMBED_DIM = 128
SEQ = 50
BATCH = 4096
ROWS_TOTAL = SEQ * BATCH
NUM_WORKERS = 32
ROWS_PER_W = ROWS_TOTAL // NUM_WORKERS
IDX_COLS = 128
IDX_ROWS_W = ROWS_PER_W // IDX_COLS
KROW = 2
CHUNK = KROW * IDX_COLS
NCHUNK = ROWS_PER_W // CHUNK
NBUF = 3


def _sc_embed(idx3d, table):
    mesh = plsc.VectorSubcoreMesh(core_axis_name="c", subcore_axis_name="s")

    @functools.partial(
        pl.kernel,
        mesh=mesh,
        out_type=[
            jax.ShapeDtypeStruct((ROWS_TOTAL, EMBED_DIM), jnp.float32),
            jax.ShapeDtypeStruct((NUM_WORKERS, IDX_ROWS_W, IDX_COLS), jnp.int32),
        ],
        scratch_types=(
            [pltpu.VMEM((IDX_ROWS_W, IDX_COLS), jnp.int32),
             pltpu.VMEM((IDX_ROWS_W, IDX_COLS), jnp.int32)]
            + [pltpu.VMEM((CHUNK, EMBED_DIM), jnp.float32) for _ in range(NBUF)]
            + [pltpu.SemaphoreType.DMA for _ in range(2 * NBUF)]
        ),
    )
    def k(idx_hbm, table_hbm, emb_hbm, mask_hbm, idx_v, mask_v, *bufs_sems):
        rows = bufs_sems[:NBUF]
        gsem = bufs_sems[NBUF:2 * NBUF]
        osem = bufs_sems[2 * NBUF:]
        wid = lax.axis_index("s") * 2 + lax.axis_index("c")
        r0 = wid * ROWS_PER_W

        def out_slice(c):
            return emb_hbm.at[pl.ds(r0 + c * CHUNK, CHUNK)]

        def chunk_start(c, b):
            for i in range(KROW):
                pltpu.async_copy(
                    table_hbm.at[idx_v.at[c * KROW + i]],
                    rows[b].at[pl.ds(i * IDX_COLS, IDX_COLS)],
                    gsem[b],
                )

        def chunk_wait(c, b):
            pltpu.make_async_copy(out_slice(c), rows[b], gsem[b]).wait()

        def out_start(c, b):
            pltpu.async_copy(rows[b], out_slice(c), osem[b])

        def out_wait(c, b):
            pltpu.make_async_copy(rows[b], out_slice(c), osem[b]).wait()

        pltpu.sync_copy(idx_hbm.at[wid], idx_v)
        for b in range(NBUF):
            chunk_start(b, b)

        def mask_row(r, carry):
            for c in range(IDX_COLS // 16):
                v = idx_v[r, pl.ds(c * 16, 16)]
                mask_v[r, pl.ds(c * 16, 16)] = jnp.minimum(
                    jnp.abs(v), jnp.full((16,), 1, jnp.int32)
                )
            return carry

        lax.fori_loop(0, IDX_ROWS_W, mask_row, 0)
        pltpu.sync_copy(mask_v, mask_hbm.at[wid])

        def outer(g, carry):
            for b in range(NBUF):
                c = g * NBUF + b
                chunk_wait(c, b)
                out_start(c, b)
                pb = (b - 1) % NBUF
                pc = c + NBUF - 1

                @pl.when((c > 0) & (pc < NCHUNK))
                def _():
                    out_wait(c - 1, pb)
                    chunk_start(pc, pb)

            return carry

        lax.fori_loop(0, (NCHUNK - 1) // NBUF, outer, 0)
        last_b = (NCHUNK - 1) % NBUF
        chunk_wait(NCHUNK - 1, last_b)
        out_start(NCHUNK - 1, last_b)
        out_wait(NCHUNK - 2, (last_b - 1) % NBUF)
        out_wait(NCHUNK - 1, last_b)

    return k(idx3d, table)


def kernel(sen_idx, table):
    idx3d = sen_idx.astype(jnp.int32).T.reshape(
        NUM_WORKERS, IDX_ROWS_W, IDX_COLS
    )
    emb, mask_i32 = _sc_embed(idx3d, table)
    sen_emb = emb.reshape(SEQ, BATCH, EMBED_DIM).transpose(1, 0, 2)
    mask = (mask_i32.reshape(SEQ, BATCH) != 0).T
    return (sen_emb, mask)


# speedup vs baseline: 1.0466x; 1.0466x over previous
import functools

import jax
import jax.numpy as jnp
from jax import lax
from jax.experimental import pallas as pl
from jax.experimental.pallas import tpu as pltpu
from jax.experimental.pallas import tpu_sc as plsc

EMBED_DIM = 128
SEQ = 50
BATCH = 4096
NUM_WORKERS = 32
BLK = BATCH // NUM_WORKERS
NBUF = 6
MAIN = (SEQ // NBUF) * NBUF


def _sc_embed(idx_t, table):
    mesh = plsc.VectorSubcoreMesh(core_axis_name="c", subcore_axis_name="s")

    @functools.partial(
        pl.kernel,
        mesh=mesh,
        out_type=[
            jax.ShapeDtypeStruct((SEQ, BATCH, EMBED_DIM), jnp.float32),
            jax.ShapeDtypeStruct((SEQ, BATCH), jnp.int32),
        ],
        scratch_types=(
            [pltpu.VMEM((SEQ, BLK), jnp.int32),
             pltpu.VMEM((SEQ, BLK), jnp.int32)]
            + [pltpu.VMEM((BLK, EMBED_DIM), jnp.float32) for _ in range(NBUF)]
            + [pltpu.SemaphoreType.DMA for _ in range(2 * NBUF)]
        ),
    )
    def k(idx_hbm, table_hbm, emb_hbm, mask_hbm, idx_v, mask_v, *bufs_sems):
        rows = bufs_sems[:NBUF]
        gsem = bufs_sems[NBUF:2 * NBUF]
        osem = bufs_sems[2 * NBUF:]
        wid = lax.axis_index("s") * 2 + lax.axis_index("c")
        n0 = wid * BLK

        def gather_start(p, b):
            pltpu.async_copy(table_hbm.at[idx_v.at[p]], rows[b], gsem[b])

        def gather_wait(p, b):
            pltpu.make_async_copy(
                table_hbm.at[idx_v.at[p]], rows[b], gsem[b]
            ).wait()

        def out_start(p, b):
            pltpu.async_copy(rows[b], emb_hbm.at[p, pl.ds(n0, BLK)], osem[b])

        def out_wait(p, b):
            pltpu.make_async_copy(
                rows[b], emb_hbm.at[p, pl.ds(n0, BLK)], osem[b]
            ).wait()

        pltpu.sync_copy(idx_hbm.at[pl.ds(0, SEQ), pl.ds(n0, BLK)], idx_v)
        for b in range(NBUF):
            gather_start(b, b)

        def mask_row(r, carry):
            for c in range(BLK // 16):
                v = idx_v[r, pl.ds(c * 16, 16)]
                mask_v[r, pl.ds(c * 16, 16)] = jnp.minimum(
                    jnp.abs(v), jnp.full((16,), 1, jnp.int32)
                )
            return carry

        lax.fori_loop(0, SEQ, mask_row, 0)
        pltpu.sync_copy(mask_v, mask_hbm.at[pl.ds(0, SEQ), pl.ds(n0, BLK)])

        def outer(t, carry):
            for b in range(NBUF):
                p = t * NBUF + b
                gather_wait(p, b)
                out_start(p, b)
                pb = (b - 1) % NBUF
                pp = p + NBUF - 1

                @pl.when((p > 0) & (pp < SEQ))
                def _():
                    out_wait(p - 1, pb)
                    gather_start(pp, pb)

            return carry

        lax.fori_loop(0, SEQ // NBUF, outer, 0)
        for c in range(MAIN, SEQ):
            b = c % NBUF
            gather_wait(c, b)
            out_start(c, b)
            out_wait(c - 1, (b - 1) % NBUF)
        out_wait(SEQ - 1, (SEQ - 1) % NBUF)

    return k(idx_t, table)


def kernel(sen_idx, table):
    idx_t = sen_idx.astype(jnp.int32).T
    emb, mask_i32 = _sc_embed(idx_t, table)
    sen_emb = emb.transpose(1, 0, 2)
    mask = (mask_i32 != 0).T
    return (sen_emb, mask)


# speedup vs baseline: 1.0518x; 1.0049x over previous
import functools

import jax
import jax.numpy as jnp
from jax import lax
from jax.experimental import pallas as pl
from jax.experimental.pallas import tpu as pltpu
from jax.experimental.pallas import tpu_sc as plsc

EMBED_DIM = 128
SEQ = 50
BATCH = 4096
NUM_WORKERS = 32
BLK = BATCH // NUM_WORKERS
NBUF = 6
MAIN = (SEQ // NBUF) * NBUF


def _sc_embed(idx_t, table):
    mesh = plsc.VectorSubcoreMesh(core_axis_name="c", subcore_axis_name="s")

    @functools.partial(
        pl.kernel,
        mesh=mesh,
        out_type=[
            jax.ShapeDtypeStruct((SEQ, BATCH, EMBED_DIM), jnp.float32),
            jax.ShapeDtypeStruct((SEQ, BATCH), jnp.int32),
        ],
        scratch_types=(
            [pltpu.VMEM((SEQ, BLK), jnp.int32),
             pltpu.VMEM((SEQ, BLK), jnp.int32)]
            + [pltpu.VMEM((BLK, EMBED_DIM), jnp.float32) for _ in range(NBUF)]
            + [pltpu.SemaphoreType.DMA for _ in range(2 * NBUF)]
        ),
    )
    def k(idx_hbm, table_hbm, emb_hbm, mask_hbm, idx_v, mask_v, *bufs_sems):
        rows = bufs_sems[:NBUF]
        gsem = bufs_sems[NBUF:2 * NBUF]
        osem = bufs_sems[2 * NBUF:]
        wid = lax.axis_index("s") * 2 + lax.axis_index("c")
        n0 = wid * BLK

        def gather_start(p, b):
            pltpu.async_copy(table_hbm.at[idx_v.at[p]], rows[b], gsem[b])

        def gather_wait(p, b):
            pltpu.make_async_copy(
                table_hbm.at[idx_v.at[p]], rows[b], gsem[b]
            ).wait()

        def out_start(p, b):
            pltpu.async_copy(rows[b], emb_hbm.at[p, pl.ds(n0, BLK)], osem[b])

        def out_wait(p, b):
            pltpu.make_async_copy(
                rows[b], emb_hbm.at[p, pl.ds(n0, BLK)], osem[b]
            ).wait()

        pltpu.sync_copy(idx_hbm.at[pl.ds(0, SEQ), pl.ds(n0, BLK)], idx_v)
        for b in range(NBUF):
            gather_start(b, b)

        def mask_row(r):
            for c in range(BLK // 16):
                v = idx_v[r, pl.ds(c * 16, 16)]
                mask_v[r, pl.ds(c * 16, 16)] = jnp.minimum(
                    jnp.abs(v), jnp.full((16,), 1, jnp.int32)
                )

        def outer(t, carry):
            for b in range(NBUF):
                p = t * NBUF + b
                mask_row(p)
                gather_wait(p, b)
                out_start(p, b)
                pb = (b - 1) % NBUF
                pp = p + NBUF - 1

                @pl.when((p > 0) & (pp < SEQ))
                def _():
                    out_wait(p - 1, pb)
                    gather_start(pp, pb)

            return carry

        lax.fori_loop(0, SEQ // NBUF, outer, 0)
        for c in range(MAIN, SEQ):
            b = c % NBUF
            mask_row(c)
            gather_wait(c, b)
            out_start(c, b)
            out_wait(c - 1, (b - 1) % NBUF)
        pltpu.sync_copy(mask_v, mask_hbm.at[pl.ds(0, SEQ), pl.ds(n0, BLK)])
        out_wait(SEQ - 1, (SEQ - 1) % NBUF)

    return k(idx_t, table)


def kernel(sen_idx, table):
    idx_t = sen_idx.astype(jnp.int32).T
    emb, mask_i32 = _sc_embed(idx_t, table)
    sen_emb = emb.transpose(1, 0, 2)
    mask = (mask_i32 != 0).T
    return (sen_emb, mask)
